# Initial kernel scaffold; baseline (speedup 1.0000x reference)
import jax
import jax.numpy as jnp
from jax.experimental import pallas as pl


def kernel(im_inds, obj_fmaps, obj_labels, Wproj, bproj, W3w, b3w, W3u, b3u, W4w, b4w, W4u, b4u, W5w, b5w, W5u, b5u, Wout, bout, Wcls, bcls):
    raise NotImplementedError("write your pallas kernel here")


# speedup vs baseline: 24.6733x; 24.6733x over previous
import ml_dtypes
import numpy as np
import jax
import jax.numpy as jnp
from jax.experimental import pallas as pl
from jax.experimental.pallas import tpu as pltpu

_N_OBJ = 256
_N_IM = 4
_C = 151
_H = 512
_OUT = 512

_INV_C = np.float32(np.float32(1.0 / _C).astype(ml_dtypes.bfloat16))


def _b16(v):
    return v.astype(jnp.bfloat16)


def _bdot(a, b, dims=((1,), (1,))):
    return jax.lax.dot_general(_b16(a), _b16(b), (dims, ((), ())),
                               preferred_element_type=jnp.float32)


def _hdot(a, b, dims=((1,), (1,))):
    return jax.lax.dot_general(a, b, (dims, ((), ())),
                               preferred_element_type=jnp.float32,
                               precision=jax.lax.Precision.HIGHEST)


def _ggnn_kernel(im_inds_ref, obj_ref, Wproj_ref, bproj_ref,
                 W3w_ref, b3w_ref, W3u_ref, b3u_ref,
                 W4w_ref, b4w_ref,
                 W5w_ref, b5w_ref, W5u_ref, b5u_ref,
                 Wout_ref, bout_ref, Wcls_ref, bcls_ref,
                 out_ref, o_scr, acc_scr):
    c = pl.program_id(0)
    nc = pl.num_programs(0)

    @pl.when(c == 0)
    def _compute_gru():
        x = _bdot(obj_ref[...], Wproj_ref[...]) + bproj_ref[0, :]
        inds = im_inds_ref[...]
        h = x
        for _ in range(3):
            hs = jnp.zeros_like(h)
            for im in range(_N_IM):
                m = inds == im
                s_im = jnp.sum(jnp.where(m, h, 0.0), axis=0, keepdims=True)
                hs = hs + jnp.where(m, s_im, 0.0)
            diff = hs - h
            a = jnp.float32(_C) * (_INV_C * _b16(diff).astype(jnp.float32))
            hU = _bdot(h, W3u_ref[...]) + b3u_ref[0, :]
            zv = jax.nn.sigmoid(_bdot(a, W3w_ref[:, :_H]) + _bdot(a, W3w_ref[:, _H:])
                                + b3w_ref[0, :] + hU)
            rv = jax.nn.sigmoid(_bdot(a, W4w_ref[:, :_H]) + _bdot(a, W4w_ref[:, _H:])
                                + b4w_ref[0, :] + hU)
            hv = jnp.tanh(_bdot(a, W5w_ref[:, :_H]) + _bdot(a, W5w_ref[:, _H:])
                          + b5w_ref[0, :] + _bdot(rv * h, W5u_ref[...]) + b5u_ref[0, :])
            h = (1.0 - zv) * h + zv * hv
        o = _bdot(h, Wout_ref[:, :_H]) + _bdot(x, Wout_ref[:, _H:]) + bout_ref[0, :]
        o_scr[...] = _b16(jnp.maximum(o, 0.0)).astype(jnp.float32)

    @pl.when(c == 0)
    def _init_acc():
        acc_scr[...] = _b16(Wcls_ref[...]).astype(jnp.float32)

    @pl.when(c > 0)
    def _accum():
        acc_scr[...] += _b16(Wcls_ref[...]).astype(jnp.float32)

    @pl.when(c == nc - 1)
    def _final():
        out_ref[...] = _hdot(o_scr[...], acc_scr[...]) + bcls_ref[0, :]


def kernel(im_inds, obj_fmaps, obj_labels, Wproj, bproj, W3w, b3w, W3u, b3u,
           W4w, b4w, W4u, b4u, W5w, b5w, W5u, b5u, Wout, bout, Wcls, bcls):
    del obj_labels, W4u, b4u
    full = lambda shape: pl.BlockSpec(shape, lambda c: tuple(0 for _ in shape))
    row = lambda v: v.reshape(1, -1)
    return pl.pallas_call(
        _ggnn_kernel,
        grid=(_C,),
        in_specs=[
            full((_N_OBJ, 1)),
            full((_N_OBJ, 4096)),
            full((_H, 4096)),
            full((1, _H)),
            full((_H, 2 * _H)), full((1, _H)),
            full((_H, _H)), full((1, _H)),
            full((_H, 2 * _H)), full((1, _H)),
            full((_H, 2 * _H)), full((1, _H)),
            full((_H, _H)), full((1, _H)),
            full((_OUT, 2 * _H)), full((1, _OUT)),
            pl.BlockSpec((_C, _OUT), lambda c: (0, c)),
            full((1, _C)),
        ],
        out_specs=pl.BlockSpec((_N_OBJ, _C), lambda c: (0, 0)),
        out_shape=jax.ShapeDtypeStruct((_N_OBJ, _C), jnp.float32),
        scratch_shapes=[
            pltpu.VMEM((_N_OBJ, _OUT), jnp.float32),
            pltpu.VMEM((_C, _OUT), jnp.float32),
        ],
    )(im_inds.reshape(_N_OBJ, 1), obj_fmaps, Wproj, row(bproj),
      W3w, row(b3w), W3u, row(b3u), W4w, row(b4w),
      W5w, row(b5w), W5u, row(b5u), Wout, row(bout), Wcls, row(bcls))


# speedup vs baseline: 73.1174x; 2.9634x over previous
import ml_dtypes
import numpy as np
import jax
import jax.numpy as jnp
from jax.experimental import pallas as pl
from jax.experimental.pallas import tpu as pltpu

_N_OBJ = 256
_N_IM = 4
_C = 151
_H = 512
_OUT = 512

_INV_C = np.float32(np.float32(1.0 / _C).astype(ml_dtypes.bfloat16))
_SLICES = 16
_NB = -(-_C // _SLICES)
_TAIL = _C - (_NB - 1) * _SLICES


def _b16(v):
    return v.astype(jnp.bfloat16)


def _bdot(a, b, dims=((1,), (1,))):
    return jax.lax.dot_general(_b16(a), _b16(b), (dims, ((), ())),
                               preferred_element_type=jnp.float32)


def _hdot(a, b, dims=((1,), (1,))):
    return jax.lax.dot_general(a, b, (dims, ((), ())),
                               preferred_element_type=jnp.float32,
                               precision=jax.lax.Precision.HIGHEST)


def _ggnn_kernel(im_inds_ref, obj_ref, Wproj_ref, bproj_ref,
                 W3w_ref, b3w_ref, W3u_ref, b3u_ref,
                 W4w_ref, b4w_ref,
                 W5w_ref, b5w_ref, W5u_ref, b5u_ref,
                 Wout_ref, bout_ref, Wcls_ref, bcls_ref,
                 out_ref, o_scr, acc_scr):
    c = pl.program_id(0)
    nc = pl.num_programs(0)

    @pl.when(c == 0)
    def _compute_gru():
        x = _bdot(obj_ref[...], Wproj_ref[...]) + bproj_ref[0, :]
        inds = im_inds_ref[...]
        h = x
        for _ in range(3):
            hs = jnp.zeros_like(h)
            for im in range(_N_IM):
                m = inds == im
                s_im = jnp.sum(jnp.where(m, h, 0.0), axis=0, keepdims=True)
                hs = hs + jnp.where(m, s_im, 0.0)
            diff = hs - h
            a = jnp.float32(_C) * (_INV_C * _b16(diff).astype(jnp.float32))
            hU = _bdot(h, W3u_ref[...]) + b3u_ref[0, :]
            zv = jax.nn.sigmoid(_bdot(a, W3w_ref[:, :_H]) + _bdot(a, W3w_ref[:, _H:])
                                + b3w_ref[0, :] + hU)
            rv = jax.nn.sigmoid(_bdot(a, W4w_ref[:, :_H]) + _bdot(a, W4w_ref[:, _H:])
                                + b4w_ref[0, :] + hU)
            hv = jnp.tanh(_bdot(a, W5w_ref[:, :_H]) + _bdot(a, W5w_ref[:, _H:])
                          + b5w_ref[0, :] + _bdot(rv * h, W5u_ref[...]) + b5u_ref[0, :])
            h = (1.0 - zv) * h + zv * hv
        o = _bdot(h, Wout_ref[:, :_H]) + _bdot(x, Wout_ref[:, _H:]) + bout_ref[0, :]
        o_scr[...] = _b16(jnp.maximum(o, 0.0)).astype(jnp.float32)

    @pl.when(c == 0)
    def _init_acc():
        acc = _b16(Wcls_ref[:, :_OUT]).astype(jnp.float32)
        for j in range(1, _SLICES):
            acc += _b16(Wcls_ref[:, j * _OUT:(j + 1) * _OUT]).astype(jnp.float32)
        acc_scr[...] = acc

    @pl.when((c > 0) & (c < nc - 1))
    def _accum():
        acc = acc_scr[...]
        for j in range(_SLICES):
            acc += _b16(Wcls_ref[:, j * _OUT:(j + 1) * _OUT]).astype(jnp.float32)
        acc_scr[...] = acc

    @pl.when(c == nc - 1)
    def _accum_tail():
        acc = acc_scr[...]
        for j in range(_TAIL):
            acc += _b16(Wcls_ref[:, j * _OUT:(j + 1) * _OUT]).astype(jnp.float32)
        acc_scr[...] = acc

    @pl.when(c == nc - 1)
    def _final():
        out_ref[...] = _hdot(o_scr[...], acc_scr[...]) + bcls_ref[0, :]


def kernel(im_inds, obj_fmaps, obj_labels, Wproj, bproj, W3w, b3w, W3u, b3u,
           W4w, b4w, W4u, b4u, W5w, b5w, W5u, b5u, Wout, bout, Wcls, bcls):
    del obj_labels, W4u, b4u
    full = lambda shape: pl.BlockSpec(shape, lambda c: tuple(0 for _ in shape))
    row = lambda v: v.reshape(1, -1)
    return pl.pallas_call(
        _ggnn_kernel,
        grid=(_NB,),
        in_specs=[
            full((_N_OBJ, 1)),
            full((_N_OBJ, 4096)),
            full((_H, 4096)),
            full((1, _H)),
            full((_H, 2 * _H)), full((1, _H)),
            full((_H, _H)), full((1, _H)),
            full((_H, 2 * _H)), full((1, _H)),
            full((_H, 2 * _H)), full((1, _H)),
            full((_H, _H)), full((1, _H)),
            full((_OUT, 2 * _H)), full((1, _OUT)),
            pl.BlockSpec((_C, _SLICES * _OUT), lambda c: (0, c)),
            full((1, _C)),
        ],
        out_specs=pl.BlockSpec((_N_OBJ, _C), lambda c: (0, 0)),
        out_shape=jax.ShapeDtypeStruct((_N_OBJ, _C), jnp.float32),
        scratch_shapes=[
            pltpu.VMEM((_N_OBJ, _OUT), jnp.float32),
            pltpu.VMEM((_C, _OUT), jnp.float32),
        ],
    )(im_inds.reshape(_N_OBJ, 1), obj_fmaps, Wproj, row(bproj),
      W3w, row(b3w), W3u, row(b3u), W4w, row(b4w),
      W5w, row(b5w), W5u, row(b5u), Wout, row(bout), Wcls, row(bcls))


# speedup vs baseline: 74.8817x; 1.0241x over previous
import ml_dtypes
import numpy as np
import jax
import jax.numpy as jnp
from jax.experimental import pallas as pl
from jax.experimental.pallas import tpu as pltpu

_N_OBJ = 256
_N_IM = 4
_C = 151
_H = 512
_OUT = 512

_INV_C = np.float32(np.float32(1.0 / _C).astype(ml_dtypes.bfloat16))
_SLICES = 32
_NB = -(-_C // _SLICES)
_TAIL = _C - (_NB - 1) * _SLICES


def _b16(v):
    return v.astype(jnp.bfloat16)


def _bdot(a, b, dims=((1,), (1,))):
    return jax.lax.dot_general(_b16(a), _b16(b), (dims, ((), ())),
                               preferred_element_type=jnp.float32)


def _hdot(a, b, dims=((1,), (1,))):
    return jax.lax.dot_general(a, b, (dims, ((), ())),
                               preferred_element_type=jnp.float32,
                               precision=jax.lax.Precision.HIGHEST)


def _ggnn_kernel(im_inds_ref, obj_ref, Wproj_ref, bproj_ref,
                 W3w_ref, b3w_ref, W3u_ref, b3u_ref,
                 W4w_ref, b4w_ref,
                 W5w_ref, b5w_ref, W5u_ref, b5u_ref,
                 Wout_ref, bout_ref, Wcls_ref, bcls_ref,
                 out_ref, o_scr, acc_scr):
    c = pl.program_id(0)
    nc = pl.num_programs(0)

    @pl.when(c == 0)
    def _compute_gru():
        x = _bdot(obj_ref[...], Wproj_ref[...]) + bproj_ref[0, :]
        inds = im_inds_ref[...]
        h = x
        for _ in range(3):
            hs = jnp.zeros_like(h)
            for im in range(_N_IM):
                m = inds == im
                s_im = jnp.sum(jnp.where(m, h, 0.0), axis=0, keepdims=True)
                hs = hs + jnp.where(m, s_im, 0.0)
            diff = hs - h
            a = jnp.float32(_C) * (_INV_C * _b16(diff).astype(jnp.float32))
            hU = _bdot(h, W3u_ref[...]) + b3u_ref[0, :]
            zv = jax.nn.sigmoid(_bdot(a, W3w_ref[:, :_H]) + _bdot(a, W3w_ref[:, _H:])
                                + b3w_ref[0, :] + hU)
            rv = jax.nn.sigmoid(_bdot(a, W4w_ref[:, :_H]) + _bdot(a, W4w_ref[:, _H:])
                                + b4w_ref[0, :] + hU)
            hv = jnp.tanh(_bdot(a, W5w_ref[:, :_H]) + _bdot(a, W5w_ref[:, _H:])
                          + b5w_ref[0, :] + _bdot(rv * h, W5u_ref[...]) + b5u_ref[0, :])
            h = (1.0 - zv) * h + zv * hv
        o = _bdot(h, Wout_ref[:, :_H]) + _bdot(x, Wout_ref[:, _H:]) + bout_ref[0, :]
        o_scr[...] = _b16(jnp.maximum(o, 0.0)).astype(jnp.float32)

    @pl.when(c == 0)
    def _init_acc():
        acc = _b16(Wcls_ref[:, :_OUT]).astype(jnp.float32)
        for j in range(1, _SLICES):
            acc += _b16(Wcls_ref[:, j * _OUT:(j + 1) * _OUT]).astype(jnp.float32)
        acc_scr[...] = acc

    @pl.when((c > 0) & (c < nc - 1))
    def _accum():
        acc = acc_scr[...]
        for j in range(_SLICES):
            acc += _b16(Wcls_ref[:, j * _OUT:(j + 1) * _OUT]).astype(jnp.float32)
        acc_scr[...] = acc

    @pl.when(c == nc - 1)
    def _accum_tail():
        acc = acc_scr[...]
        for j in range(_TAIL):
            acc += _b16(Wcls_ref[:, j * _OUT:(j + 1) * _OUT]).astype(jnp.float32)
        acc_scr[...] = acc

    @pl.when(c == nc - 1)
    def _final():
        out_ref[...] = _hdot(o_scr[...], acc_scr[...]) + bcls_ref[0, :]


def kernel(im_inds, obj_fmaps, obj_labels, Wproj, bproj, W3w, b3w, W3u, b3u,
           W4w, b4w, W4u, b4u, W5w, b5w, W5u, b5u, Wout, bout, Wcls, bcls):
    del obj_labels, W4u, b4u
    full = lambda shape: pl.BlockSpec(shape, lambda c: tuple(0 for _ in shape))
    row = lambda v: v.reshape(1, -1)
    return pl.pallas_call(
        _ggnn_kernel,
        grid=(_NB,),
        in_specs=[
            full((_N_OBJ, 1)),
            full((_N_OBJ, 4096)),
            full((_H, 4096)),
            full((1, _H)),
            full((_H, 2 * _H)), full((1, _H)),
            full((_H, _H)), full((1, _H)),
            full((_H, 2 * _H)), full((1, _H)),
            full((_H, 2 * _H)), full((1, _H)),
            full((_H, _H)), full((1, _H)),
            full((_OUT, 2 * _H)), full((1, _OUT)),
            pl.BlockSpec((_C, _SLICES * _OUT), lambda c: (0, c)),
            full((1, _C)),
        ],
        out_specs=pl.BlockSpec((_N_OBJ, _C), lambda c: (0, 0)),
        out_shape=jax.ShapeDtypeStruct((_N_OBJ, _C), jnp.float32),
        scratch_shapes=[
            pltpu.VMEM((_N_OBJ, _OUT), jnp.float32),
            pltpu.VMEM((_C, _OUT), jnp.float32),
        ],
    )(im_inds.reshape(_N_OBJ, 1), obj_fmaps, Wproj, row(bproj),
      W3w, row(b3w), W3u, row(b3u), W4w, row(b4w),
      W5w, row(b5w), W5u, row(b5u), Wout, row(bout), Wcls, row(bcls))


# speedup vs baseline: 75.9911x; 1.0148x over previous
import ml_dtypes
import numpy as np
import jax
import jax.numpy as jnp
from jax.experimental import pallas as pl
from jax.experimental.pallas import tpu as pltpu

_N_OBJ = 256
_N_IM = 4
_C = 151
_H = 512
_OUT = 512

_INV_C = np.float32(np.float32(1.0 / _C).astype(ml_dtypes.bfloat16))
_SLICES = 16
_NB = 5
_TAIL_B = _C - 80 - (_NB - 1) * _SLICES


def _b16(v):
    return v.astype(jnp.bfloat16)


def _bdot(a, b, dims=((1,), (1,))):
    return jax.lax.dot_general(_b16(a), _b16(b), (dims, ((), ())),
                               preferred_element_type=jnp.float32)


def _hdot(a, b, dims=((1,), (1,))):
    return jax.lax.dot_general(a, b, (dims, ((), ())),
                               preferred_element_type=jnp.float32,
                               precision=jax.lax.Precision.HIGHEST)


def _ggnn_kernel(im_inds_ref, obj_ref, Wproj_ref, bproj_ref,
                 W3w_ref, b3w_ref, W3u_ref, b3u_ref,
                 W4w_ref, b4w_ref,
                 W5w_ref, b5w_ref, W5u_ref, b5u_ref,
                 Wout_ref, bout_ref, WclsA_ref, WclsB_ref, bcls_ref,
                 out_ref, o_scr, acc_scr):
    c = pl.program_id(0)
    nc = pl.num_programs(0)

    @pl.when(c == 0)
    def _compute_gru():
        x = _bdot(obj_ref[...], Wproj_ref[...]) + bproj_ref[0, :]
        inds = im_inds_ref[...]
        h = x
        for _ in range(3):
            hs = jnp.zeros_like(h)
            for im in range(_N_IM):
                m = inds == im
                s_im = jnp.sum(jnp.where(m, h, 0.0), axis=0, keepdims=True)
                hs = hs + jnp.where(m, s_im, 0.0)
            diff = hs - h
            a = jnp.float32(_C) * (_INV_C * _b16(diff).astype(jnp.float32))
            hU = _bdot(h, W3u_ref[...]) + b3u_ref[0, :]
            zv = jax.nn.sigmoid(_bdot(a, W3w_ref[:, :_H]) + _bdot(a, W3w_ref[:, _H:])
                                + b3w_ref[0, :] + hU)
            rv = jax.nn.sigmoid(_bdot(a, W4w_ref[:, :_H]) + _bdot(a, W4w_ref[:, _H:])
                                + b4w_ref[0, :] + hU)
            hv = jnp.tanh(_bdot(a, W5w_ref[:, :_H]) + _bdot(a, W5w_ref[:, _H:])
                          + b5w_ref[0, :] + _bdot(rv * h, W5u_ref[...]) + b5u_ref[0, :])
            h = (1.0 - zv) * h + zv * hv
        o = _bdot(h, Wout_ref[:, :_H]) + _bdot(x, Wout_ref[:, _H:]) + bout_ref[0, :]
        o_scr[...] = _b16(jnp.maximum(o, 0.0)).astype(jnp.float32)

    @pl.when(c == 0)
    def _init_acc():
        acc = _b16(WclsA_ref[:, :_OUT]).astype(jnp.float32)
        for j in range(1, _SLICES):
            acc += _b16(WclsA_ref[:, j * _OUT:(j + 1) * _OUT]).astype(jnp.float32)
        for j in range(_SLICES):
            acc += _b16(WclsB_ref[:, j * _OUT:(j + 1) * _OUT]).astype(jnp.float32)
        acc_scr[...] = acc

    @pl.when((c > 0) & (c < nc - 1))
    def _accum():
        acc = acc_scr[...]
        for j in range(_SLICES):
            acc += _b16(WclsA_ref[:, j * _OUT:(j + 1) * _OUT]).astype(jnp.float32)
        for j in range(_SLICES):
            acc += _b16(WclsB_ref[:, j * _OUT:(j + 1) * _OUT]).astype(jnp.float32)
        acc_scr[...] = acc

    @pl.when(c == nc - 1)
    def _accum_tail():
        acc = acc_scr[...]
        for j in range(_SLICES):
            acc += _b16(WclsA_ref[:, j * _OUT:(j + 1) * _OUT]).astype(jnp.float32)
        for j in range(_TAIL_B):
            acc += _b16(WclsB_ref[:, j * _OUT:(j + 1) * _OUT]).astype(jnp.float32)
        acc_scr[...] = acc

    @pl.when(c == nc - 1)
    def _final():
        out_ref[...] = _hdot(o_scr[...], acc_scr[...]) + bcls_ref[0, :]


def kernel(im_inds, obj_fmaps, obj_labels, Wproj, bproj, W3w, b3w, W3u, b3u,
           W4w, b4w, W4u, b4u, W5w, b5w, W5u, b5u, Wout, bout, Wcls, bcls):
    del obj_labels, W4u, b4u
    full = lambda shape: pl.BlockSpec(shape, lambda c: tuple(0 for _ in shape))
    row = lambda v: v.reshape(1, -1)
    return pl.pallas_call(
        _ggnn_kernel,
        grid=(_NB,),
        in_specs=[
            full((_N_OBJ, 1)),
            full((_N_OBJ, 4096)),
            full((_H, 4096)),
            full((1, _H)),
            full((_H, 2 * _H)), full((1, _H)),
            full((_H, _H)), full((1, _H)),
            full((_H, 2 * _H)), full((1, _H)),
            full((_H, 2 * _H)), full((1, _H)),
            full((_H, _H)), full((1, _H)),
            full((_OUT, 2 * _H)), full((1, _OUT)),
            pl.BlockSpec((_C, _SLICES * _OUT), lambda c: (0, c)),
            pl.BlockSpec((_C, _SLICES * _OUT), lambda c: (0, c + 5)),
            full((1, _C)),
        ],
        out_specs=pl.BlockSpec((_N_OBJ, _C), lambda c: (0, 0)),
        out_shape=jax.ShapeDtypeStruct((_N_OBJ, _C), jnp.float32),
        scratch_shapes=[
            pltpu.VMEM((_N_OBJ, _OUT), jnp.float32),
            pltpu.VMEM((_C, _OUT), jnp.float32),
        ],
    )(im_inds.reshape(_N_OBJ, 1), obj_fmaps, Wproj, row(bproj),
      W3w, row(b3w), W3u, row(b3u), W4w, row(b4w),
      W5w, row(b5w), W5u, row(b5u), Wout, row(bout), Wcls, Wcls, row(bcls))


# speedup vs baseline: 77.6195x; 1.0214x over previous
import ml_dtypes
import numpy as np
import jax
import jax.numpy as jnp
from jax.experimental import pallas as pl
from jax.experimental.pallas import tpu as pltpu

_N_OBJ = 256
_N_IM = 4
_C = 151
_H = 512
_OUT = 512

_INV_C = np.float32(np.float32(1.0 / _C).astype(ml_dtypes.bfloat16))
_SLICES = 16
_NB = 5
_TAIL_B = _C - 80 - (_NB - 1) * _SLICES


def _b16(v):
    return v.astype(jnp.bfloat16)


def _bdot(a, b, dims=((1,), (1,))):
    return jax.lax.dot_general(_b16(a), _b16(b), (dims, ((), ())),
                               preferred_element_type=jnp.float32)


def _hdot(a, b, dims=((1,), (1,))):
    return jax.lax.dot_general(a, b, (dims, ((), ())),
                               preferred_element_type=jnp.float32,
                               precision=jax.lax.Precision.HIGHEST)


def _ggnn_kernel(im_inds_ref, obj_ref, Wproj_ref, bproj_ref,
                 W3w_ref, b3w_ref, W3u_ref, b3u_ref,
                 W4w_ref, b4w_ref,
                 W5w_ref, b5w_ref, W5u_ref, b5u_ref,
                 Wout_ref, bout_ref, WclsA_ref, WclsB_ref, bcls_ref,
                 out_ref, o_scr, acc_scr):
    c = pl.program_id(0)
    nc = pl.num_programs(0)

    @pl.when(c == 0)
    def _compute_gru():
        x = _bdot(obj_ref[...], Wproj_ref[...]) + bproj_ref[0, :]
        inds = im_inds_ref[...]
        h = x
        for _ in range(3):
            hs = jnp.zeros_like(h)
            for im in range(_N_IM):
                m = inds == im
                s_im = jnp.sum(jnp.where(m, h, 0.0), axis=0, keepdims=True)
                hs = hs + jnp.where(m, s_im, 0.0)
            diff = hs - h
            a = jnp.float32(_C) * (_INV_C * _b16(diff).astype(jnp.float32))
            hU = _bdot(h, W3u_ref[...]) + b3u_ref[0, :]
            zv = jax.nn.sigmoid(_bdot(a, W3w_ref[:, :_H]) + _bdot(a, W3w_ref[:, _H:])
                                + b3w_ref[0, :] + hU)
            rv = jax.nn.sigmoid(_bdot(a, W4w_ref[:, :_H]) + _bdot(a, W4w_ref[:, _H:])
                                + b4w_ref[0, :] + hU)
            hv = jnp.tanh(_bdot(a, W5w_ref[:, :_H]) + _bdot(a, W5w_ref[:, _H:])
                          + b5w_ref[0, :] + _bdot(rv * h, W5u_ref[...]) + b5u_ref[0, :])
            h = (1.0 - zv) * h + zv * hv
        o = _bdot(h, Wout_ref[:, :_H]) + _bdot(x, Wout_ref[:, _H:]) + bout_ref[0, :]
        o_scr[...] = _b16(jnp.maximum(o, 0.0)).astype(jnp.float32)

    @pl.when(c == 0)
    def _init_acc():
        acc = WclsA_ref[:, :_OUT]
        for j in range(1, _SLICES):
            acc += WclsA_ref[:, j * _OUT:(j + 1) * _OUT]
        for j in range(_SLICES):
            acc += WclsB_ref[:, j * _OUT:(j + 1) * _OUT]
        acc_scr[...] = acc

    @pl.when((c > 0) & (c < nc - 1))
    def _accum():
        acc = acc_scr[...]
        for j in range(_SLICES):
            acc += WclsA_ref[:, j * _OUT:(j + 1) * _OUT]
        for j in range(_SLICES):
            acc += WclsB_ref[:, j * _OUT:(j + 1) * _OUT]
        acc_scr[...] = acc

    @pl.when(c == nc - 1)
    def _accum_tail():
        acc = acc_scr[...]
        for j in range(_SLICES):
            acc += WclsA_ref[:, j * _OUT:(j + 1) * _OUT]
        for j in range(_TAIL_B):
            acc += WclsB_ref[:, j * _OUT:(j + 1) * _OUT]
        acc_scr[...] = acc

    @pl.when(c == nc - 1)
    def _final():
        out_ref[...] = _hdot(o_scr[...], acc_scr[...]) + bcls_ref[0, :]


def kernel(im_inds, obj_fmaps, obj_labels, Wproj, bproj, W3w, b3w, W3u, b3u,
           W4w, b4w, W4u, b4u, W5w, b5w, W5u, b5u, Wout, bout, Wcls, bcls):
    del obj_labels, W4u, b4u
    full = lambda shape: pl.BlockSpec(shape, lambda c: tuple(0 for _ in shape))
    row = lambda v: v.reshape(1, -1)
    return pl.pallas_call(
        _ggnn_kernel,
        grid=(_NB,),
        in_specs=[
            full((_N_OBJ, 1)),
            full((_N_OBJ, 4096)),
            full((_H, 4096)),
            full((1, _H)),
            full((_H, 2 * _H)), full((1, _H)),
            full((_H, _H)), full((1, _H)),
            full((_H, 2 * _H)), full((1, _H)),
            full((_H, 2 * _H)), full((1, _H)),
            full((_H, _H)), full((1, _H)),
            full((_OUT, 2 * _H)), full((1, _OUT)),
            pl.BlockSpec((_C, _SLICES * _OUT), lambda c: (0, c)),
            pl.BlockSpec((_C, _SLICES * _OUT), lambda c: (0, c + 5)),
            full((1, _C)),
        ],
        out_specs=pl.BlockSpec((_N_OBJ, _C), lambda c: (0, 0)),
        out_shape=jax.ShapeDtypeStruct((_N_OBJ, _C), jnp.float32),
        scratch_shapes=[
            pltpu.VMEM((_N_OBJ, _OUT), jnp.float32),
            pltpu.VMEM((_C, _OUT), jnp.float32),
        ],
    )(im_inds.reshape(_N_OBJ, 1), obj_fmaps, Wproj, row(bproj),
      W3w, row(b3w), W3u, row(b3u), W4w, row(b4w),
      W5w, row(b5w), W5u, row(b5u), Wout, row(bout), Wcls, Wcls, row(bcls))
